# baseline (speedup 1.0000x reference)
import dataclasses
import functools

import jax
import jax.numpy as jnp
from jax.experimental import pallas as pl
from jax.experimental.pallas import tpu as pltpu
from jax.experimental.pallas import tpu_sc as plsc

N_NODES = 10000
N_EDGES = 320000
D_NODE = 128
D_EDGE = 16
D_OUT = 128

NC = 2
NS = 16
NW = NC * NS
EPT = N_EDGES // NW
CHUNK = 40
NCHUNK = EPT // CHUNK
NROWCHUNK = N_NODES // CHUNK
CNT_ROWS = 80
LANES = 16

_HIGH = jax.lax.Precision.HIGHEST


def _proj_body(nf_ref, wn_ref, bn_ref, ef_ref, wm_ref,
               p_ref, q_ref, r_ref, e_ref):
    z = jnp.dot(nf_ref[...], wn_ref[...], precision=_HIGH) + bn_ref[...]
    p_ref[...] = z[:, 0:128]
    q_ref[...] = z[:, 128:256]
    r_ref[...] = z[:, 256:384]
    e_ref[...] = jax.lax.dot_general(
        ef_ref[...].astype(jnp.bfloat16), wm_ref[...],
        (((0,), (0,)), ((), ())), preferred_element_type=jnp.float32)


def _tc_proj(nf, wn, bn, ef, wm):
    grid = 25
    nb = N_NODES // grid
    eb = N_EDGES // grid
    return pl.pallas_call(
        _proj_body,
        grid=(grid,),
        in_specs=[
            pl.BlockSpec((nb, D_NODE), lambda i: (i, 0)),
            pl.BlockSpec((D_NODE, 384), lambda i: (0, 0)),
            pl.BlockSpec((1, 384), lambda i: (0, 0)),
            pl.BlockSpec((D_EDGE, eb), lambda i: (0, i)),
            pl.BlockSpec((D_EDGE, D_OUT), lambda i: (0, 0)),
        ],
        out_specs=[
            pl.BlockSpec((nb, D_OUT), lambda i: (i, 0)),
            pl.BlockSpec((nb, D_OUT), lambda i: (i, 0)),
            pl.BlockSpec((nb, D_OUT), lambda i: (i, 0)),
            pl.BlockSpec((eb, D_OUT), lambda i: (i, 0)),
        ],
        out_shape=[
            jax.ShapeDtypeStruct((N_NODES, D_OUT), jnp.float32),
            jax.ShapeDtypeStruct((N_NODES, D_OUT), jnp.float32),
            jax.ShapeDtypeStruct((N_NODES, D_OUT), jnp.float32),
            jax.ShapeDtypeStruct((N_EDGES, D_OUT), jnp.float32),
        ],
    )(nf, wn, bn, ef, wm)


def _sc_edge_body(p_hbm, q_hbm, e_hbm, src_hbm, dst_hbm,
                  out_s_hbm, out_c_hbm,
                  acc, acc_cnt,
                  src_a, dst_a, src_b, dst_b, cidx_v, col_v,
                  p_a, q_a, m_a, p_b, q_b, m_b, oh_v,
                  sem_sa, sem_da, sem_sb, sem_db,
                  sem_pa, sem_qa, sem_ea, sem_pb, sem_qb, sem_eb,
                  sem_ma, sem_mb, sem_o):
    c = jax.lax.axis_index("c")
    s = jax.lax.axis_index("s")
    wid = s * NC + c
    tbase = wid * EPT

    zero16 = jnp.zeros((LANES,), jnp.float32)
    one16 = jnp.ones((LANES,), jnp.float32)
    iota16 = jax.lax.iota(jnp.int32, LANES)
    mask_hi = iota16 >= 8

    @pl.loop(0, CHUNK)
    def _(r):
        for seg in range(D_OUT // LANES):
            sl = pl.ds(seg * LANES, LANES)
            m_a[r, sl] = zero16
            oh_v[r, sl] = zero16
            oh_v[r + CHUNK, sl] = zero16

    @pl.loop(0, (NROWCHUNK + NS - 1) // NS)
    def _(j):
        k = j * NS + s

        @pl.when(k < NROWCHUNK)
        def _():
            pltpu.sync_copy(m_a, acc.at[pl.ds(k * CHUNK, CHUNK)])

    @pl.when(s == 0)
    def _():
        pltpu.sync_copy(m_a, acc_cnt.at[pl.ds(0, CHUNK)])
        pltpu.sync_copy(m_a, acc_cnt.at[pl.ds(CHUNK, CHUNK)])

    plsc.subcore_barrier()

    def issue_idx(i, src_v, dst_v, sem_s, sem_d):
        eb = tbase + i * CHUNK
        pltpu.async_copy(src_hbm.at[pl.ds(eb, CHUNK)], src_v, sem_s)
        pltpu.async_copy(dst_hbm.at[pl.ds(eb, CHUNK)], dst_v, sem_d)

    def wait_idx(src_v, dst_v, sem_s, sem_d):
        pltpu.make_async_copy(src_hbm.at[pl.ds(0, CHUNK)], src_v, sem_s).wait()
        pltpu.make_async_copy(dst_hbm.at[pl.ds(0, CHUNK)], dst_v, sem_d).wait()

    def issue_gathers(i, src_v, dst_v, p_v, q_v, m_v, sem_p, sem_q, sem_e):
        eb = tbase + i * CHUNK
        pltpu.async_copy(p_hbm.at[src_v], p_v, sem_p)
        pltpu.async_copy(q_hbm.at[dst_v], q_v, sem_q)
        pltpu.async_copy(e_hbm.at[pl.ds(eb, CHUNK)], m_v, sem_e)

    def wait_gathers(src_v, dst_v, p_v, q_v, m_v, sem_p, sem_q, sem_e):
        pltpu.make_async_copy(p_hbm.at[src_v], p_v, sem_p).wait()
        pltpu.make_async_copy(q_hbm.at[dst_v], q_v, sem_q).wait()
        pltpu.make_async_copy(e_hbm.at[pl.ds(0, CHUNK)], m_v, sem_e).wait()

    def compute(p_v, q_v, m_v):
        @pl.loop(0, CHUNK)
        def _(r):
            for seg in range(D_OUT // LANES):
                sl = pl.ds(seg * LANES, LANES)
                m_v[r, sl] = jnp.maximum(
                    p_v[r, sl] + q_v[r, sl] + m_v[r, sl], 0.0)

    oh_groups = [(0, None), (16, None), (24, mask_hi)]

    def build_counts(dst_v, base):
        for off, msk in oh_groups:
            d16 = dst_v[pl.ds(off, LANES)]
            row16 = iota16 + (base + off)
            col16 = jax.lax.bitwise_and(d16, 127)
            col_v[pl.ds(base + off, LANES)] = col16
            cidx_v[pl.ds(base + off, LANES)] = jax.lax.shift_right_logical(
                d16, 7)
            plsc.store_scatter(oh_v, [row16, col16], one16, mask=msk)

    def clear_counts():
        for base in (0, CHUNK):
            for off, msk in oh_groups:
                row16 = iota16 + (base + off)
                col16 = col_v[pl.ds(base + off, LANES)]
                plsc.store_scatter(oh_v, [row16, col16], zero16, mask=msk)

    def wait_scatter(src_v, dst_ref, sem):
        pltpu.make_async_copy(src_v, dst_ref, sem).wait()

    issue_idx(0, src_a, dst_a, sem_sa, sem_da)
    issue_idx(1, src_b, dst_b, sem_sb, sem_db)
    wait_idx(src_a, dst_a, sem_sa, sem_da)
    issue_gathers(0, src_a, dst_a, p_a, q_a, m_a, sem_pa, sem_qa, sem_ea)

    nbody = NCHUNK // 2

    @pl.loop(0, nbody)
    def _(j):
        i = j * 2
        wait_idx(src_b, dst_b, sem_sb, sem_db)
        issue_gathers(i + 1, src_b, dst_b, p_b, q_b, m_b,
                      sem_pb, sem_qb, sem_eb)
        wait_gathers(src_a, dst_a, p_a, q_a, m_a, sem_pa, sem_qa, sem_ea)
        compute(p_a, q_a, m_a)
        pltpu.async_copy(m_a, acc.at[dst_a], sem_ma, add=True)
        build_counts(dst_a, 0)
        wait_gathers(src_b, dst_b, p_b, q_b, m_b, sem_pb, sem_qb, sem_eb)
        compute(p_b, q_b, m_b)
        pltpu.async_copy(m_b, acc.at[dst_b], sem_mb, add=True)
        build_counts(dst_b, CHUNK)
        pltpu.async_copy(oh_v, acc_cnt.at[cidx_v], sem_o, add=True)
        wait_scatter(m_a, acc.at[dst_a], sem_ma)

        @pl.when(j < nbody - 1)
        def _():
            issue_idx(i + 2, src_a, dst_a, sem_sa, sem_da)

        wait_scatter(oh_v, acc_cnt.at[cidx_v], sem_o)
        clear_counts()
        wait_scatter(m_b, acc.at[dst_b], sem_mb)

        @pl.when(j < nbody - 1)
        def _():
            issue_idx(i + 3, src_b, dst_b, sem_sb, sem_db)
            wait_idx(src_a, dst_a, sem_sa, sem_da)
            issue_gathers(i + 2, src_a, dst_a, p_a, q_a, m_a,
                          sem_pa, sem_qa, sem_ea)

    plsc.subcore_barrier()

    @pl.loop(0, (NROWCHUNK + NS - 1) // NS)
    def _(j):
        k = j * NS + s

        @pl.when(k < NROWCHUNK)
        def _():
            pltpu.sync_copy(acc.at[pl.ds(k * CHUNK, CHUNK)],
                            out_s_hbm.at[c, pl.ds(k * CHUNK, CHUNK)])

    @pl.when(s == 0)
    def _():
        pltpu.sync_copy(acc_cnt, out_c_hbm.at[c])


def _sc_edge(p, q, e, src, dst):
    mesh = plsc.VectorSubcoreMesh(core_axis_name="c", subcore_axis_name="s")
    cp = pltpu.CompilerParams()
    if "needs_layout_passes" in pltpu.CompilerParams.__dataclass_fields__:
        cp = dataclasses.replace(cp, needs_layout_passes=False)
    cp = dataclasses.replace(cp, use_tc_tiling_on_sc=True)
    kern = pl.kernel(
        _sc_edge_body,
        out_type=[
            jax.ShapeDtypeStruct((NC, N_NODES, D_OUT), jnp.float32),
            jax.ShapeDtypeStruct((NC, CNT_ROWS, 128), jnp.float32),
        ],
        mesh=mesh,
        scratch_types=[
            pltpu.VMEM_SHARED((N_NODES, D_OUT), jnp.float32),
            pltpu.VMEM_SHARED((CNT_ROWS, 128), jnp.float32),
        ] + [pltpu.VMEM((CHUNK,), jnp.int32)] * 4 + [
            pltpu.VMEM((2 * CHUNK,), jnp.int32),
            pltpu.VMEM((2 * CHUNK,), jnp.int32),
            pltpu.VMEM((CHUNK, D_OUT), jnp.float32),
            pltpu.VMEM((CHUNK, D_OUT), jnp.float32),
            pltpu.VMEM((CHUNK, D_OUT), jnp.float32),
            pltpu.VMEM((CHUNK, D_OUT), jnp.float32),
            pltpu.VMEM((CHUNK, D_OUT), jnp.float32),
            pltpu.VMEM((CHUNK, D_OUT), jnp.float32),
            pltpu.VMEM((2 * CHUNK, D_OUT), jnp.float32),
        ] + [pltpu.SemaphoreType.DMA] * 13,
        compiler_params=cp,
    )
    return kern(p, q, e, src, dst)


def _finish_body(parts_ref, cnt_ref, r_ref, wv_ref, out_ref):
    s = parts_ref[0] + parts_ref[1]
    neigh = s / jnp.maximum(cnt_ref[...], 1.0)
    out_ref[...] = jax.nn.relu(
        jnp.dot(neigh, wv_ref[...], precision=_HIGH) + r_ref[...])


def _tc_finish(parts, cnt, r, wv):
    grid = 10
    nb = N_NODES // grid
    return pl.pallas_call(
        _finish_body,
        grid=(grid,),
        in_specs=[
            pl.BlockSpec((NC, nb, D_OUT), lambda i: (0, i, 0)),
            pl.BlockSpec((nb, 1), lambda i: (i, 0)),
            pl.BlockSpec((nb, D_OUT), lambda i: (i, 0)),
            pl.BlockSpec((D_OUT, D_OUT), lambda i: (0, 0)),
        ],
        out_specs=pl.BlockSpec((nb, D_OUT), lambda i: (i, 0)),
        out_shape=jax.ShapeDtypeStruct((N_NODES, D_OUT), jnp.float32),
    )(parts, cnt, r, wv)


def kernel(node_feats, edge_feats, edge_index, W_e, b_e, W_v, b_v):
    wn = jnp.concatenate([W_e[0:128], W_e[144:272], W_v[128:256]], axis=1)
    bn = jnp.concatenate(
        [b_e, jnp.zeros((128,), jnp.float32), b_v]).reshape(1, 384)
    wm = W_e[128:144].astype(jnp.bfloat16)

    p, q, r, e = _tc_proj(node_feats, wn, bn, edge_feats.T, wm)

    src = edge_index[0].astype(jnp.int32)
    dst = edge_index[1].astype(jnp.int32)

    parts, parts_cnt = _sc_edge(p, q, e, src, dst)
    cnt = (parts_cnt[0] + parts_cnt[1]).reshape(-1)[:N_NODES]
    return _tc_finish(parts, cnt.reshape(N_NODES, 1), r, W_v[0:128])


# speedup vs baseline: 1.0190x; 1.0190x over previous
import dataclasses
import functools

import jax
import jax.numpy as jnp
from jax.experimental import pallas as pl
from jax.experimental.pallas import tpu as pltpu
from jax.experimental.pallas import tpu_sc as plsc

N_NODES = 10000
N_EDGES = 320000
D_NODE = 128
D_EDGE = 16
D_OUT = 128

NC = 2
NS = 16
NW = NC * NS
EPT = N_EDGES // NW
CHUNK = 40
NCHUNK = EPT // CHUNK
NROWCHUNK = N_NODES // CHUNK
CNT_ROWS = 80
LANES = 16

_HIGH = jax.lax.Precision.HIGHEST


def _proj_body(nf_ref, wn_ref, bn_ref, ef_ref, wm_ref,
               p_ref, q_ref, r_ref, e_ref):
    z = jnp.dot(nf_ref[...], wn_ref[...], precision=_HIGH) + bn_ref[...]
    p_ref[...] = z[:, 0:128]
    q_ref[...] = z[:, 128:256]
    r_ref[...] = z[:, 256:384]
    e_ref[...] = jax.lax.dot_general(
        ef_ref[...].astype(jnp.bfloat16), wm_ref[...],
        (((0,), (0,)), ((), ())), preferred_element_type=jnp.float32)


def _tc_proj(nf, wn, bn, ef, wm):
    grid = 25
    nb = N_NODES // grid
    eb = N_EDGES // grid
    return pl.pallas_call(
        _proj_body,
        grid=(grid,),
        in_specs=[
            pl.BlockSpec((nb, D_NODE), lambda i: (i, 0)),
            pl.BlockSpec((D_NODE, 384), lambda i: (0, 0)),
            pl.BlockSpec((1, 384), lambda i: (0, 0)),
            pl.BlockSpec((D_EDGE, eb), lambda i: (0, i)),
            pl.BlockSpec((D_EDGE, D_OUT), lambda i: (0, 0)),
        ],
        out_specs=[
            pl.BlockSpec((nb, D_OUT), lambda i: (i, 0)),
            pl.BlockSpec((nb, D_OUT), lambda i: (i, 0)),
            pl.BlockSpec((nb, D_OUT), lambda i: (i, 0)),
            pl.BlockSpec((eb, D_OUT), lambda i: (i, 0)),
        ],
        out_shape=[
            jax.ShapeDtypeStruct((N_NODES, D_OUT), jnp.float32),
            jax.ShapeDtypeStruct((N_NODES, D_OUT), jnp.float32),
            jax.ShapeDtypeStruct((N_NODES, D_OUT), jnp.float32),
            jax.ShapeDtypeStruct((N_EDGES, D_OUT), jnp.float32),
        ],
    )(nf, wn, bn, ef, wm)


def _sc_edge_body(p_hbm, q_hbm, e_hbm, src_hbm, dst_hbm,
                  out_s_hbm, out_c_hbm,
                  acc, acc_cnt,
                  src_a, dst_a, src_b, dst_b, cidx_a, cidx_b, col_a, col_b,
                  p_a, q_a, m_a, oh_a, p_b, q_b, m_b, oh_b,
                  sem_sa, sem_da, sem_sb, sem_db,
                  sem_pa, sem_qa, sem_ea, sem_pb, sem_qb, sem_eb,
                  sem_ma, sem_mb, sem_oa, sem_ob):
    c = jax.lax.axis_index("c")
    s = jax.lax.axis_index("s")
    wid = s * NC + c
    tbase = wid * EPT

    zero16 = jnp.zeros((LANES,), jnp.float32)
    one16 = jnp.ones((LANES,), jnp.float32)
    iota16 = jax.lax.iota(jnp.int32, LANES)
    mask_hi = iota16 >= 8

    @pl.loop(0, CHUNK)
    def _(r):
        for seg in range(D_OUT // LANES):
            sl = pl.ds(seg * LANES, LANES)
            m_a[r, sl] = zero16
            oh_a[r, sl] = zero16
            oh_b[r, sl] = zero16

    @pl.loop(0, (NROWCHUNK + NS - 1) // NS)
    def _(j):
        k = j * NS + s

        @pl.when(k < NROWCHUNK)
        def _():
            pltpu.sync_copy(m_a, acc.at[pl.ds(k * CHUNK, CHUNK)])

    @pl.when(s == 0)
    def _():
        pltpu.sync_copy(m_a, acc_cnt.at[pl.ds(0, CHUNK)])
        pltpu.sync_copy(m_a, acc_cnt.at[pl.ds(CHUNK, CHUNK)])

    plsc.subcore_barrier()

    def issue_idx(i, src_v, dst_v, sem_s, sem_d):
        eb = tbase + i * CHUNK
        pltpu.async_copy(src_hbm.at[pl.ds(eb, CHUNK)], src_v, sem_s)
        pltpu.async_copy(dst_hbm.at[pl.ds(eb, CHUNK)], dst_v, sem_d)

    def wait_idx(src_v, dst_v, sem_s, sem_d):
        pltpu.make_async_copy(src_hbm.at[pl.ds(0, CHUNK)], src_v, sem_s).wait()
        pltpu.make_async_copy(dst_hbm.at[pl.ds(0, CHUNK)], dst_v, sem_d).wait()

    def issue_gathers(i, src_v, dst_v, p_v, q_v, m_v, sem_p, sem_q, sem_e):
        eb = tbase + i * CHUNK
        pltpu.async_copy(p_hbm.at[src_v], p_v, sem_p)
        pltpu.async_copy(q_hbm.at[dst_v], q_v, sem_q)
        pltpu.async_copy(e_hbm.at[pl.ds(eb, CHUNK)], m_v, sem_e)

    def wait_gathers(src_v, dst_v, p_v, q_v, m_v, sem_p, sem_q, sem_e):
        pltpu.make_async_copy(p_hbm.at[src_v], p_v, sem_p).wait()
        pltpu.make_async_copy(q_hbm.at[dst_v], q_v, sem_q).wait()
        pltpu.make_async_copy(e_hbm.at[pl.ds(0, CHUNK)], m_v, sem_e).wait()

    def compute(p_v, q_v, m_v):
        @pl.loop(0, CHUNK)
        def _(r):
            for seg in range(D_OUT // LANES):
                sl = pl.ds(seg * LANES, LANES)
                m_v[r, sl] = jnp.maximum(
                    p_v[r, sl] + q_v[r, sl] + m_v[r, sl], 0.0)

    oh_groups = [(0, None), (16, None), (24, mask_hi)]

    def build_counts(dst_v, cidx_v, col_v, oh_v):
        for off, msk in oh_groups:
            d16 = dst_v[pl.ds(off, LANES)]
            row16 = iota16 + off
            col16 = jax.lax.bitwise_and(d16, 127)
            col_v[pl.ds(off, LANES)] = col16
            cidx_v[pl.ds(off, LANES)] = jax.lax.shift_right_logical(d16, 7)
            plsc.store_scatter(oh_v, [row16, col16], one16, mask=msk)

    def clear_counts(col_v, oh_v):
        for off, msk in oh_groups:
            row16 = iota16 + off
            col16 = col_v[pl.ds(off, LANES)]
            plsc.store_scatter(oh_v, [row16, col16], zero16, mask=msk)

    def wait_scatter(src_v, dst_ref, sem):
        pltpu.make_async_copy(src_v, dst_ref, sem).wait()

    issue_idx(0, src_a, dst_a, sem_sa, sem_da)
    issue_idx(1, src_b, dst_b, sem_sb, sem_db)
    wait_idx(src_a, dst_a, sem_sa, sem_da)
    issue_gathers(0, src_a, dst_a, p_a, q_a, m_a, sem_pa, sem_qa, sem_ea)

    nbody = NCHUNK // 2

    @pl.loop(0, nbody)
    def _(j):
        i = j * 2
        wait_idx(src_b, dst_b, sem_sb, sem_db)
        issue_gathers(i + 1, src_b, dst_b, p_b, q_b, m_b,
                      sem_pb, sem_qb, sem_eb)
        wait_gathers(src_a, dst_a, p_a, q_a, m_a, sem_pa, sem_qa, sem_ea)
        compute(p_a, q_a, m_a)
        pltpu.async_copy(m_a, acc.at[dst_a], sem_ma, add=True)
        build_counts(dst_a, cidx_a, col_a, oh_a)
        pltpu.async_copy(oh_a, acc_cnt.at[cidx_a], sem_oa, add=True)
        wait_gathers(src_b, dst_b, p_b, q_b, m_b, sem_pb, sem_qb, sem_eb)
        compute(p_b, q_b, m_b)
        pltpu.async_copy(m_b, acc.at[dst_b], sem_mb, add=True)
        build_counts(dst_b, cidx_b, col_b, oh_b)
        pltpu.async_copy(oh_b, acc_cnt.at[cidx_b], sem_ob, add=True)
        wait_scatter(oh_a, acc_cnt.at[cidx_a], sem_oa)
        clear_counts(col_a, oh_a)
        wait_scatter(m_a, acc.at[dst_a], sem_ma)

        @pl.when(j < nbody - 1)
        def _():
            issue_idx(i + 2, src_a, dst_a, sem_sa, sem_da)

        wait_scatter(oh_b, acc_cnt.at[cidx_b], sem_ob)
        clear_counts(col_b, oh_b)
        wait_scatter(m_b, acc.at[dst_b], sem_mb)

        @pl.when(j < nbody - 1)
        def _():
            issue_idx(i + 3, src_b, dst_b, sem_sb, sem_db)
            wait_idx(src_a, dst_a, sem_sa, sem_da)
            issue_gathers(i + 2, src_a, dst_a, p_a, q_a, m_a,
                          sem_pa, sem_qa, sem_ea)

    plsc.subcore_barrier()

    @pl.loop(0, (NROWCHUNK + NS - 1) // NS)
    def _(j):
        k = j * NS + s

        @pl.when(k < NROWCHUNK)
        def _():
            pltpu.sync_copy(acc.at[pl.ds(k * CHUNK, CHUNK)],
                            out_s_hbm.at[c, pl.ds(k * CHUNK, CHUNK)])

    @pl.when(s == 0)
    def _():
        pltpu.sync_copy(acc_cnt, out_c_hbm.at[c])


def _sc_edge(p, q, e, src, dst):
    mesh = plsc.VectorSubcoreMesh(core_axis_name="c", subcore_axis_name="s")
    cp = pltpu.CompilerParams()
    if "needs_layout_passes" in pltpu.CompilerParams.__dataclass_fields__:
        cp = dataclasses.replace(cp, needs_layout_passes=False)
    cp = dataclasses.replace(cp, use_tc_tiling_on_sc=True)
    kern = pl.kernel(
        _sc_edge_body,
        out_type=[
            jax.ShapeDtypeStruct((NC, N_NODES, D_OUT), jnp.float32),
            jax.ShapeDtypeStruct((NC, CNT_ROWS, 128), jnp.float32),
        ],
        mesh=mesh,
        scratch_types=[
            pltpu.VMEM_SHARED((N_NODES, D_OUT), jnp.float32),
            pltpu.VMEM_SHARED((CNT_ROWS, 128), jnp.float32),
        ] + [pltpu.VMEM((CHUNK,), jnp.int32)] * 8 + [
            pltpu.VMEM((CHUNK, D_OUT), jnp.float32),
            pltpu.VMEM((CHUNK, D_OUT), jnp.float32),
            pltpu.VMEM((CHUNK, D_OUT), jnp.float32),
            pltpu.VMEM((CHUNK, D_OUT), jnp.float32),
            pltpu.VMEM((CHUNK, D_OUT), jnp.float32),
            pltpu.VMEM((CHUNK, D_OUT), jnp.float32),
            pltpu.VMEM((CHUNK, D_OUT), jnp.float32),
            pltpu.VMEM((CHUNK, D_OUT), jnp.float32),
        ] + [pltpu.SemaphoreType.DMA] * 14,
        compiler_params=cp,
    )
    return kern(p, q, e, src, dst)


def _finish_body(parts_ref, cnt_ref, r_ref, wv_ref, out_ref):
    s = parts_ref[0] + parts_ref[1]
    neigh = s / jnp.maximum(cnt_ref[...], 1.0)
    out_ref[...] = jax.nn.relu(
        jnp.dot(neigh, wv_ref[...], precision=_HIGH) + r_ref[...])


def _tc_finish(parts, cnt, r, wv):
    grid = 10
    nb = N_NODES // grid
    return pl.pallas_call(
        _finish_body,
        grid=(grid,),
        in_specs=[
            pl.BlockSpec((NC, nb, D_OUT), lambda i: (0, i, 0)),
            pl.BlockSpec((nb, 1), lambda i: (i, 0)),
            pl.BlockSpec((nb, D_OUT), lambda i: (i, 0)),
            pl.BlockSpec((D_OUT, D_OUT), lambda i: (0, 0)),
        ],
        out_specs=pl.BlockSpec((nb, D_OUT), lambda i: (i, 0)),
        out_shape=jax.ShapeDtypeStruct((N_NODES, D_OUT), jnp.float32),
    )(parts, cnt, r, wv)


def kernel(node_feats, edge_feats, edge_index, W_e, b_e, W_v, b_v):
    wn = jnp.concatenate([W_e[0:128], W_e[144:272], W_v[128:256]], axis=1)
    bn = jnp.concatenate(
        [b_e, jnp.zeros((128,), jnp.float32), b_v]).reshape(1, 384)
    wm = W_e[128:144].astype(jnp.bfloat16)

    p, q, r, e = _tc_proj(node_feats, wn, bn, edge_feats.T, wm)

    src = edge_index[0].astype(jnp.int32)
    dst = edge_index[1].astype(jnp.int32)

    parts, parts_cnt = _sc_edge(p, q, e, src, dst)
    cnt = (parts_cnt[0] + parts_cnt[1]).reshape(-1)[:N_NODES]
    return _tc_finish(parts, cnt.reshape(N_NODES, 1), r, W_v[0:128])
